# baseline (speedup 1.0000x reference)
import jax
import jax.numpy as jnp
from jax.experimental import pallas as pl
from jax.experimental.pallas import tpu as pltpu


def _mm_kernel(x_ref, w_ref, ap_ref, h_ref, al_ref):
    h = jnp.dot(x_ref[...], w_ref[...], preferred_element_type=jnp.float32)
    h_ref[...] = h
    al_ref[...] = jnp.dot(h, ap_ref[...], preferred_element_type=jnp.float32)


def kernel(x, edge_index, W, a):
    n, f_in = x.shape
    f_out = W.shape[1]
    a_pair = a.reshape(2, f_out).T
    h, al = pl.pallas_call(
        _mm_kernel,
        out_shape=(
            jax.ShapeDtypeStruct((n, f_out), jnp.float32),
            jax.ShapeDtypeStruct((n, 2), jnp.float32),
        ),
    )(x, W, a_pair)
    src = edge_index[0]
    dst = edge_index[1]
    s = al[src, 0] + al[dst, 1]
    w = jnp.exp(-jnp.tanh(s))
    rowsum = jax.ops.segment_sum(w, src, num_segments=n) + 1e-8
    hp = jax.ops.segment_sum(w[:, None] * jnp.take(h, dst, axis=0), src,
                             num_segments=n)
    return hp / rowsum[:, None]


# speedup vs baseline: 21.0538x; 21.0538x over previous
import jax
import jax.numpy as jnp
from jax import lax
from jax.experimental import pallas as pl
from jax.experimental.pallas import tpu as pltpu
from jax.experimental.pallas import tpu_sc as plsc

N = 10000
E = 320000
F = 128
NC = 2
NS = 16
NW = NC * NS
EDGES_PER_TILE = E // NW
EBLK = 2000
NBLK = EDGES_PER_TILE // EBLK
BCHUNKS = EBLK // 16
NP_ = 10240
ROWS_PER_TILE = NP_ // NS
ZROWS = 32
GRID = 10


def _mm_kernel(x_ref, w_ref, ap_ref, h_ref, al_ref):
    h = jnp.dot(x_ref[...], w_ref[...], preferred_element_type=jnp.float32)
    h_ref[...] = h
    al_ref[...] = jnp.dot(h, ap_ref[...], preferred_element_type=jnp.float32)


def _combine_kernel(p0_ref, p1_ref, rs_ref, o_ref):
    den = jnp.sum(rs_ref[...], axis=1, keepdims=True) + 1e-8
    o_ref[...] = (p0_ref[...] + p1_ref[...]) / den


def _sc_edge_kernel(h_hbm, src_hbm, dst_hbm, als_hbm, ald_hbm,
                    out_hbm, rs_hbm,
                    als_v, ald_v, src_v, dst_v, zero_v, grows_v, srows_v,
                    rs_v, tmp_s, tmp_w, acc_sh, gsem, ssem):
    cid = lax.axis_index("c")
    sid = lax.axis_index("s")
    wid = cid * NS + sid

    pltpu.sync_copy(als_hbm, als_v)
    pltpu.sync_copy(ald_hbm, ald_v)
    ebase = wid * EDGES_PER_TILE

    zv = jnp.zeros((16,), jnp.float32)

    def zrs_body(i, _):
        rs_v[pl.ds(i * 16, 16)] = zv
        return ()

    lax.fori_loop(0, NP_ // 16, zrs_body, (), unroll=False)

    def zrow_body(r, _):
        for i in range(F // 16):
            zero_v[r, pl.ds(i * 16, 16)] = zv
        return ()

    lax.fori_loop(0, ZROWS, zrow_body, (), unroll=False)
    for z in range(ROWS_PER_TILE // ZROWS):
        pltpu.sync_copy(zero_v, acc_sh.at[pl.ds(sid * ROWS_PER_TILE + z * ZROWS, ZROWS)])
    plsc.subcore_barrier()

    lanes = lax.iota(jnp.int32, 16)

    def chunk_body(c, _):
        s16 = src_v[pl.ds(c * 16, 16)]
        d16 = dst_v[pl.ds(c * 16, 16)]
        s = plsc.load_gather(als_v, [s16]) + plsc.load_gather(ald_v, [d16])
        t = jnp.exp(-2.0 * jnp.abs(s))
        tanh_s = jnp.sign(s) * (1.0 - t) / (1.0 + t)
        w = jnp.exp(-tanh_s)
        pltpu.async_copy(h_hbm.at[d16], grows_v, gsem).wait()
        for r in range(16):
            wr = w[r]
            for i in range(F // 16):
                srows_v[r, pl.ds(i * 16, 16)] = grows_v[r, pl.ds(i * 16, 16)] * wr
        pltpu.async_copy(srows_v, acc_sh.at[s16], ssem, add=True).wait()
        tmp_s[...] = s16
        tmp_w[...] = w
        total = w
        seen_before = lanes < 0
        for k in range(1, 16):
            idx = (lanes + k) & 15
            eq = plsc.load_gather(tmp_s, [idx]) == s16
            total = total + jnp.where(eq, plsc.load_gather(tmp_w, [idx]), 0.0)
            seen_before = seen_before | (eq & (lanes >= 16 - k))
        plsc.addupdate_scatter(rs_v, [s16], total, mask=~seen_before)
        return ()

    for b in range(NBLK):
        pltpu.sync_copy(src_hbm.at[pl.ds(ebase + b * EBLK, EBLK)], src_v)
        pltpu.sync_copy(dst_hbm.at[pl.ds(ebase + b * EBLK, EBLK)], dst_v)
        lax.fori_loop(0, BCHUNKS, chunk_body, (), unroll=False)
    plsc.subcore_barrier()

    obase = cid * NP_ + sid * ROWS_PER_TILE
    pltpu.sync_copy(acc_sh.at[pl.ds(sid * ROWS_PER_TILE, ROWS_PER_TILE)],
                    out_hbm.at[pl.ds(obase, ROWS_PER_TILE)])
    pltpu.sync_copy(rs_v, rs_hbm.at[wid])


@jax.jit
def _sc_edge(h, src, dst, als, ald):
    mesh = plsc.VectorSubcoreMesh(core_axis_name="c", subcore_axis_name="s")
    return pl.kernel(
        _sc_edge_kernel,
        out_type=(
            jax.ShapeDtypeStruct((NC * NP_, F), jnp.float32),
            jax.ShapeDtypeStruct((NW, NP_), jnp.float32),
        ),
        mesh=mesh,
        compiler_params=pltpu.CompilerParams(needs_layout_passes=False),
        scratch_types=[
            pltpu.VMEM((NP_,), jnp.float32),
            pltpu.VMEM((NP_,), jnp.float32),
            pltpu.VMEM((EBLK,), jnp.int32),
            pltpu.VMEM((EBLK,), jnp.int32),
            pltpu.VMEM((ZROWS, F), jnp.float32),
            pltpu.VMEM((16, F), jnp.float32),
            pltpu.VMEM((16, F), jnp.float32),
            pltpu.VMEM((NP_,), jnp.float32),
            pltpu.VMEM((16,), jnp.int32),
            pltpu.VMEM((16,), jnp.float32),
            pltpu.VMEM_SHARED((NP_, F), jnp.float32),
            pltpu.SemaphoreType.DMA,
            pltpu.SemaphoreType.DMA,
        ],
    )(h, src, dst, als, ald)


def kernel(x, edge_index, W, a):
    n, f_in = x.shape
    f_out = W.shape[1]
    a_pair = a.reshape(2, f_out).T
    h, al = pl.pallas_call(
        _mm_kernel,
        out_shape=(
            jax.ShapeDtypeStruct((n, f_out), jnp.float32),
            jax.ShapeDtypeStruct((n, 2), jnp.float32),
        ),
    )(x, W, a_pair)
    als = jnp.pad(al[:, 0], (0, NP_ - n))
    ald = jnp.pad(al[:, 1], (0, NP_ - n))
    src = edge_index[0]
    dst = edge_index[1]
    part, rs = _sc_edge(h, src, dst, als, ald)
    rs_t = rs.T
    out = pl.pallas_call(
        _combine_kernel,
        grid=(GRID,),
        in_specs=[
            pl.BlockSpec((n // GRID, f_out), lambda i: (i, 0)),
            pl.BlockSpec((n // GRID, f_out), lambda i: (i, 0)),
            pl.BlockSpec((n // GRID, NW), lambda i: (i, 0)),
        ],
        out_specs=pl.BlockSpec((n // GRID, f_out), lambda i: (i, 0)),
        out_shape=jax.ShapeDtypeStruct((n, f_out), jnp.float32),
    )(part[:N], part[NP_:NP_ + N], rs_t[:N])
    return out


# speedup vs baseline: 41.4954x; 1.9709x over previous
import jax
import jax.numpy as jnp
from jax import lax
from jax.experimental import pallas as pl
from jax.experimental.pallas import tpu as pltpu
from jax.experimental.pallas import tpu_sc as plsc

N = 10000
E = 320000
F = 128
NC = 2
NS = 16
NW = NC * NS
EDGES_PER_TILE = E // NW
EBLK = 2000
NBLK = EDGES_PER_TILE // EBLK
BCHUNKS = EBLK // 16
NPAIRS = (BCHUNKS - 1) // 2
NP_ = 10240
ROWS_PER_TILE = NP_ // NS
ZROWS = 16
GRID = 10


def _mm_kernel(x_ref, w_ref, ap_ref, h_ref, al_ref):
    h = jnp.dot(x_ref[...], w_ref[...], preferred_element_type=jnp.float32)
    h_ref[...] = h
    al_ref[...] = jnp.dot(h, ap_ref[...], preferred_element_type=jnp.float32)


def _combine_kernel(p0_ref, p1_ref, rs_ref, o_ref):
    den = jnp.sum(rs_ref[...], axis=1, keepdims=True) + 1e-8
    o_ref[...] = (p0_ref[...] + p1_ref[...]) / den


def _sc_edge_kernel(h_hbm, src_hbm, dst_hbm, als_hbm, ald_hbm,
                    out_hbm, rs_hbm,
                    als_v, ald_v, src_v, dst_v, zero_v,
                    grows0, grows1, srows0, srows1,
                    rs_v, acc_sh, gsem0, gsem1, ssem0, ssem1):
    cid = lax.axis_index("c")
    sid = lax.axis_index("s")
    wid = cid * NS + sid
    grows = (grows0, grows1)
    srows = (srows0, srows1)
    gsem = (gsem0, gsem1)
    ssem = (ssem0, ssem1)

    pltpu.sync_copy(als_hbm, als_v)
    pltpu.sync_copy(ald_hbm, ald_v)
    ebase = wid * EDGES_PER_TILE

    zv = jnp.zeros((16,), jnp.float32)

    def zrs_body(i, _):
        rs_v[pl.ds(i * 16, 16)] = zv
        return ()

    lax.fori_loop(0, NP_ // 16, zrs_body, (), unroll=False)

    def zrow_body(r, _):
        for i in range(F // 16):
            zero_v[r, pl.ds(i * 16, 16)] = zv
        return ()

    lax.fori_loop(0, ZROWS, zrow_body, (), unroll=False)
    for z in range(ROWS_PER_TILE // ZROWS):
        pltpu.sync_copy(zero_v, acc_sh.at[pl.ds(sid * ROWS_PER_TILE + z * ZROWS, ZROWS)])
    plsc.subcore_barrier()

    def issue_gather(c, u):
        d16 = dst_v[pl.ds(c * 16, 16)]
        return pltpu.async_copy(h_hbm.at[d16], grows[u], gsem[u])

    def wait_gather(u):
        d16 = dst_v[pl.ds(0, 16)]
        pltpu.make_async_copy(h_hbm.at[d16], grows[u], gsem[u]).wait()

    def wait_scatter(u):
        s16 = src_v[pl.ds(0, 16)]
        pltpu.make_async_copy(srows[u], acc_sh.at[s16], ssem[u]).wait()

    def process(c, u, wait_s):
        s16 = src_v[pl.ds(c * 16, 16)]
        d16 = dst_v[pl.ds(c * 16, 16)]
        s = plsc.load_gather(als_v, [s16]) + plsc.load_gather(ald_v, [d16])
        t = jnp.exp(-2.0 * jnp.abs(s))
        tanh_s = jnp.sign(s) * (1.0 - t) / (1.0 + t)
        w = jnp.exp(-tanh_s)
        wait_gather(u)
        if wait_s:
            wait_scatter(u)
        for r in range(16):
            wr = w[r]
            for i in range(F // 16):
                srows[u][r, pl.ds(i * 16, 16)] = grows[u][r, pl.ds(i * 16, 16)] * wr
        pltpu.async_copy(srows[u], acc_sh.at[s16], ssem[u], add=True)
        plsc.addupdate_scatter(rs_v, [s16], w)

    for b in range(NBLK):
        pltpu.sync_copy(src_hbm.at[pl.ds(ebase + b * EBLK, EBLK)], src_v)
        pltpu.sync_copy(dst_hbm.at[pl.ds(ebase + b * EBLK, EBLK)], dst_v)
        issue_gather(0, 0)

        def pair_body(p, _, first_pair=False):
            c = p * 2
            issue_gather(c + 1, 1)
            process(c, 0, wait_s=not first_pair)
            issue_gather(c + 2, 0)
            process(c + 1, 1, wait_s=not first_pair)
            return ()

        if b == 0:
            pair_body(0, (), first_pair=True)
            lax.fori_loop(1, NPAIRS, pair_body, (), unroll=False)
        else:
            lax.fori_loop(0, NPAIRS, pair_body, (), unroll=False)
        process(BCHUNKS - 1, 0, wait_s=True)

    wait_scatter(0)
    wait_scatter(1)
    plsc.subcore_barrier()

    obase = cid * NP_ + sid * ROWS_PER_TILE
    pltpu.sync_copy(acc_sh.at[pl.ds(sid * ROWS_PER_TILE, ROWS_PER_TILE)],
                    out_hbm.at[pl.ds(obase, ROWS_PER_TILE)])
    pltpu.sync_copy(rs_v, rs_hbm.at[wid])


@jax.jit
def _sc_edge(h, src, dst, als, ald):
    mesh = plsc.VectorSubcoreMesh(core_axis_name="c", subcore_axis_name="s")
    return pl.kernel(
        _sc_edge_kernel,
        out_type=(
            jax.ShapeDtypeStruct((NC * NP_, F), jnp.float32),
            jax.ShapeDtypeStruct((NW, NP_), jnp.float32),
        ),
        mesh=mesh,
        compiler_params=pltpu.CompilerParams(needs_layout_passes=False),
        scratch_types=[
            pltpu.VMEM((NP_,), jnp.float32),
            pltpu.VMEM((NP_,), jnp.float32),
            pltpu.VMEM((EBLK,), jnp.int32),
            pltpu.VMEM((EBLK,), jnp.int32),
            pltpu.VMEM((ZROWS, F), jnp.float32),
            pltpu.VMEM((16, F), jnp.float32),
            pltpu.VMEM((16, F), jnp.float32),
            pltpu.VMEM((16, F), jnp.float32),
            pltpu.VMEM((16, F), jnp.float32),
            pltpu.VMEM((NP_,), jnp.float32),
            pltpu.VMEM_SHARED((NP_, F), jnp.float32),
            pltpu.SemaphoreType.DMA,
            pltpu.SemaphoreType.DMA,
            pltpu.SemaphoreType.DMA,
            pltpu.SemaphoreType.DMA,
        ],
    )(h, src, dst, als, ald)


def kernel(x, edge_index, W, a):
    n, f_in = x.shape
    f_out = W.shape[1]
    a_pair = a.reshape(2, f_out).T
    h, al = pl.pallas_call(
        _mm_kernel,
        out_shape=(
            jax.ShapeDtypeStruct((n, f_out), jnp.float32),
            jax.ShapeDtypeStruct((n, 2), jnp.float32),
        ),
    )(x, W, a_pair)
    als = jnp.pad(al[:, 0], (0, NP_ - n))
    ald = jnp.pad(al[:, 1], (0, NP_ - n))
    src = edge_index[0]
    dst = edge_index[1]
    part, rs = _sc_edge(h, src, dst, als, ald)
    rs_t = rs.T
    out = pl.pallas_call(
        _combine_kernel,
        grid=(GRID,),
        in_specs=[
            pl.BlockSpec((n // GRID, f_out), lambda i: (i, 0)),
            pl.BlockSpec((n // GRID, f_out), lambda i: (i, 0)),
            pl.BlockSpec((n // GRID, NW), lambda i: (i, 0)),
        ],
        out_specs=pl.BlockSpec((n // GRID, f_out), lambda i: (i, 0)),
        out_shape=jax.ShapeDtypeStruct((n, f_out), jnp.float32),
    )(part[:N], part[NP_:NP_ + N], rs_t[:N])
    return out
